# baseline (speedup 1.0000x reference)
import jax
import jax.numpy as jnp
from jax import lax
from jax.experimental import pallas as pl
from jax.experimental.pallas import tpu as pltpu
from jax.experimental.pallas import tpu_sc as plsc

_B, _T, _D = 4, 8192, 768

_INFO = plsc.get_sparse_core_info()
_NC = _INFO.num_cores
_NS = _INFO.num_subcores
_NW = _NC * _NS
_ROWS = _T // _NW
_CHUNK = 128
_NCHUNK = _ROWS // _CHUNK


def _sc_body(table_hbm, out_hbm, buf):
    wid = lax.axis_index("s") * _NC + lax.axis_index("c")
    base = wid * _ROWS
    for ch in range(_NCHUNK):
        row0 = base + ch * _CHUNK
        pltpu.sync_copy(table_hbm.at[pl.ds(row0, _CHUNK)], buf)
        for k in range(_B):
            b = lax.rem(wid + k, _B)
            pltpu.sync_copy(buf, out_hbm.at[b, pl.ds(row0, _CHUNK)])


def kernel(x, table):
    del x
    mesh = plsc.VectorSubcoreMesh(core_axis_name="c", subcore_axis_name="s")
    run = pl.kernel(
        _sc_body,
        mesh=mesh,
        out_type=jax.ShapeDtypeStruct((_B, _T, _D), jnp.float32),
        scratch_types=[pltpu.VMEM((_CHUNK, _D), jnp.float32)],
    )
    return run(table)
